# Initial kernel scaffold; baseline (speedup 1.0000x reference)
import jax
import jax.numpy as jnp
from jax.experimental import pallas as pl


def kernel(des, tweet, num_prop, cat_prop, edge_index, edge_type, W_des, b_des, W_tweet, b_tweet, W_num, b_num, W_cat, b_cat, W_in, b_in, root1, rel1, bias1, root2, rel2, bias2, W_o1, b_o1, W_o2, b_o2):
    raise NotImplementedError("write your pallas kernel here")


# speedup vs baseline: 3.2730x; 3.2730x over previous
import functools

import jax
import jax.numpy as jnp
from jax import lax
from jax.experimental import pallas as pl
from jax.experimental.pallas import tpu as pltpu
from jax.experimental.pallas import tpu_sc as plsc

NND = 10000
NED = 320000
NRELS = 2
HD = 128
NCORE, NSUB, LN = 2, 16, 16
NPADR = 10240
ACCR = 10368
DUMPROW = NPADR
EPT = NED // NSUB
CH = 80
NCHUNK = EPT // CH

_mesh = plsc.VectorSubcoreMesh(
    core_axis_name="c", subcore_axis_name="s", num_cores=NCORE, num_subcores=NSUB)
_sc_params = pltpu.CompilerParams(needs_layout_passes=False)


def _lrelu(v):
    return jnp.where(v >= 0, v, 0.01 * v)


@functools.partial(
    pl.kernel,
    out_type=jax.ShapeDtypeStruct((NSUB, 2 * NPADR), jnp.float32),
    mesh=_mesh,
    scratch_types=[
        pltpu.VMEM((EPT,), jnp.int32),
        pltpu.VMEM((EPT,), jnp.int32),
        pltpu.VMEM((2 * NPADR,), jnp.float32),
    ],
    compiler_params=_sc_params,
)
def _sc_counts(dst_hbm, typ_hbm, out_hbm, dstb, typb, cnt):
    cid = lax.axis_index("c")
    sid = lax.axis_index("s")
    zeros16 = jnp.zeros((LN,), jnp.float32)
    ones16 = jnp.ones((LN,), jnp.float32)

    @pl.when(cid == 0)
    def _():
        def z(i, carry):
            cnt[pl.ds(i * LN, LN)] = zeros16
            return carry
        lax.fori_loop(0, 2 * NPADR // LN, z, 0)

        base = sid * EPT
        pltpu.sync_copy(dst_hbm.at[pl.ds(base, EPT)], dstb)
        pltpu.sync_copy(typ_hbm.at[pl.ds(base, EPT)], typb)

        def acc(i, carry):
            d = dstb[pl.ds(i * LN, LN)]
            t = typb[pl.ds(i * LN, LN)]
            plsc.addupdate_scatter(cnt, [d * 2 + t], ones16)
            return carry
        lax.fori_loop(0, EPT // LN, acc, 0)

        pltpu.sync_copy(cnt, out_hbm.at[sid])


@functools.partial(
    pl.kernel,
    out_type=jax.ShapeDtypeStruct((2 * NPADR, HD), jnp.float32),
    mesh=_mesh,
    scratch_types=[
        pltpu.VMEM_SHARED((ACCR, HD), jnp.float32),
        pltpu.VMEM((CH,), jnp.int32),
        pltpu.VMEM((CH,), jnp.int32),
        pltpu.VMEM((CH,), jnp.int32),
        pltpu.VMEM((CH,), jnp.int32),
        pltpu.VMEM((CH, HD), jnp.float32),
        pltpu.VMEM((81, HD), jnp.float32),
        pltpu.SemaphoreType.DMA,
    ],
    compiler_params=_sc_params,
)
def _sc_agg(x_hbm, src_hbm, dst_hbm, typ_hbm, out_hbm,
            accum, srcb, dstb, typb, sidx, rows, zbuf, sem):
    cid = lax.axis_index("c")
    sid = lax.axis_index("s")
    zeros16 = jnp.zeros((LN,), jnp.float32)
    dump16 = jnp.full((LN,), DUMPROW, jnp.int32)

    def zrow(r, carry):
        for c in range(8):
            zbuf[r, pl.ds(c * LN, LN)] = zeros16
        return carry
    lax.fori_loop(0, 81, zrow, 0)

    def zacc(i, carry):
        pltpu.sync_copy(zbuf, accum.at[pl.ds(sid * 648 + i * 81, 81)])
        return carry
    lax.fori_loop(0, 8, zacc, 0)
    plsc.subcore_barrier()

    base = sid * EPT

    def chunk(j, carry):
        eoff = base + j * CH
        pltpu.sync_copy(src_hbm.at[pl.ds(eoff, CH)], srcb)
        pltpu.sync_copy(dst_hbm.at[pl.ds(eoff, CH)], dstb)
        pltpu.sync_copy(typ_hbm.at[pl.ds(eoff, CH)], typb)
        gather = pltpu.async_copy(x_hbm.at[srcb], rows, sem)

        def mk(v, c2):
            d = dstb[pl.ds(v * LN, LN)]
            t = typb[pl.ds(v * LN, LN)]
            sidx[pl.ds(v * LN, LN)] = jnp.where(t == cid, d, dump16)
            return c2
        lax.fori_loop(0, CH // LN, mk, 0)
        gather.wait()
        pltpu.sync_copy(rows, accum.at[sidx], add=True)
        return carry
    lax.fori_loop(0, NCHUNK, chunk, 0)
    plsc.subcore_barrier()

    rpt = NPADR // NSUB
    pltpu.sync_copy(accum.at[pl.ds(sid * rpt, rpt)],
                    out_hbm.at[pl.ds(cid * NPADR + sid * rpt, rpt)])


_BLK = 1000
_GRID = NND // _BLK


def _tc_pre_body(des_r, tw_r, np_r, cp_r, wd_r, wt_r, wn_r, wc_r,
                 bd_r, bt_r, bn_r, bc_r, win_r, bin_r, out_r):
    d = _lrelu(jnp.dot(des_r[...], wd_r[...], preferred_element_type=jnp.float32) + bd_r[...])
    t = _lrelu(jnp.dot(tw_r[...], wt_r[...], preferred_element_type=jnp.float32) + bt_r[...])
    n = _lrelu(jnp.dot(np_r[...], wn_r[...], preferred_element_type=jnp.float32) + bn_r[...])
    c = _lrelu(jnp.dot(cp_r[...], wc_r[...], preferred_element_type=jnp.float32) + bc_r[...])
    x = jnp.concatenate([d, t, n, c], axis=1)
    out_r[...] = _lrelu(jnp.dot(x, win_r[...], preferred_element_type=jnp.float32) + bin_r[...])


def _tc_pre(des, tw, npad, cpad, wd, wt, wn, wc, bd, bt, bn, bc, win, bin_):
    full = lambda s: pl.BlockSpec(s, lambda i: (0, 0))
    rows = lambda w: pl.BlockSpec((_BLK, w), lambda i: (i, 0))
    return pl.pallas_call(
        _tc_pre_body,
        grid=(_GRID,),
        in_specs=[rows(768), rows(768), rows(8), rows(8),
                  full((768, 32)), full((768, 32)), full((8, 32)), full((8, 32)),
                  full((1, 32)), full((1, 32)), full((1, 32)), full((1, 32)),
                  full((HD, HD)), full((1, HD))],
        out_specs=rows(HD),
        out_shape=jax.ShapeDtypeStruct((NND, HD), jnp.float32),
    )(des, tw, npad, cpad, wd, wt, wn, wc, bd, bt, bn, bc, win, bin_)


def _conv_out(x_r, s0_r, s1_r, cnt_r, root_r, r0_r, r1_r, bias_r):
    cnt = jnp.sum(cnt_r[...], axis=0)
    c0 = jnp.maximum(cnt[:, 0:1], 1.0)
    c1 = jnp.maximum(cnt[:, 1:2], 1.0)
    h0 = s0_r[...] / c0
    h1 = s1_r[...] / c1
    return (jnp.dot(x_r[...], root_r[...], preferred_element_type=jnp.float32)
            + bias_r[...]
            + jnp.dot(h0, r0_r[...], preferred_element_type=jnp.float32)
            + jnp.dot(h1, r1_r[...], preferred_element_type=jnp.float32))


def _tc_conv_body(x_r, s0_r, s1_r, cnt_r, root_r, r0_r, r1_r, bias_r, out_r):
    out_r[...] = _conv_out(x_r, s0_r, s1_r, cnt_r, root_r, r0_r, r1_r, bias_r)


def _tc_conv2_body(x_r, s0_r, s1_r, cnt_r, root_r, r0_r, r1_r, bias_r,
                   wo1_r, bo1_r, wo2_r, bo2_r, out_r):
    o = _conv_out(x_r, s0_r, s1_r, cnt_r, root_r, r0_r, r1_r, bias_r)
    y = _lrelu(jnp.dot(o, wo1_r[...], preferred_element_type=jnp.float32) + bo1_r[...])
    out_r[...] = jnp.dot(y, wo2_r[...], preferred_element_type=jnp.float32) + bo2_r[...]


def _tc_conv(x, s0, s1, cnt, root, r0, r1, bias):
    full = lambda s: pl.BlockSpec(s, lambda i: (0, 0))
    rows = lambda w: pl.BlockSpec((_BLK, w), lambda i: (i, 0))
    return pl.pallas_call(
        _tc_conv_body,
        grid=(_GRID,),
        in_specs=[rows(HD), rows(HD), rows(HD),
                  pl.BlockSpec((NSUB, _BLK, 2), lambda i: (0, i, 0)),
                  full((HD, HD)), full((HD, HD)), full((HD, HD)), full((1, HD))],
        out_specs=rows(HD),
        out_shape=jax.ShapeDtypeStruct((NND, HD), jnp.float32),
    )(x, s0, s1, cnt, root, r0, r1, bias)


def _tc_conv2(x, s0, s1, cnt, root, r0, r1, bias, wo1, bo1, wo2, bo2):
    full = lambda s: pl.BlockSpec(s, lambda i: (0, 0))
    rows = lambda w: pl.BlockSpec((_BLK, w), lambda i: (i, 0))
    return pl.pallas_call(
        _tc_conv2_body,
        grid=(_GRID,),
        in_specs=[rows(HD), rows(HD), rows(HD),
                  pl.BlockSpec((NSUB, _BLK, 2), lambda i: (0, i, 0)),
                  full((HD, HD)), full((HD, HD)), full((HD, HD)), full((1, HD)),
                  full((HD, HD)), full((1, HD)), full((HD, 2)), full((1, 2))],
        out_specs=rows(2),
        out_shape=jax.ShapeDtypeStruct((NND, 2), jnp.float32),
    )(x, s0, s1, cnt, root, r0, r1, bias, wo1, bo1, wo2, bo2)


def kernel(des, tweet, num_prop, cat_prop, edge_index, edge_type,
           W_des, b_des, W_tweet, b_tweet, W_num, b_num, W_cat, b_cat,
           W_in, b_in, root1, rel1, bias1, root2, rel2, bias2,
           W_o1, b_o1, W_o2, b_o2):
    src = edge_index[0]
    dst = edge_index[1]
    et = edge_type

    npad = jnp.pad(num_prop, ((0, 0), (0, 3)))
    cpad = jnp.pad(cat_prop, ((0, 0), (0, 5)))
    wn = jnp.pad(W_num, ((0, 3), (0, 0)))
    wc = jnp.pad(W_cat, ((0, 5), (0, 0)))
    r2 = lambda b: b.reshape(1, -1)

    x = _tc_pre(des, tweet, npad, cpad, W_des, W_tweet, wn, wc,
                r2(b_des), r2(b_tweet), r2(b_num), r2(b_cat), W_in, r2(b_in))

    cnt = _sc_counts(dst, et).reshape(NSUB, NPADR, 2)

    s1 = _sc_agg(x, src, dst, et).reshape(2, NPADR, HD)
    x1 = _tc_conv(x, s1[0], s1[1], cnt, root1, rel1[0], rel1[1], r2(bias1))

    s2 = _sc_agg(x1, src, dst, et).reshape(2, NPADR, HD)
    out = _tc_conv2(x1, s2[0], s2[1], cnt, root2, rel2[0], rel2[1], r2(bias2),
                    W_o1, r2(b_o1), W_o2, r2(b_o2))
    return out
